# baseline (speedup 1.0000x reference)
import math

import jax
import jax.numpy as jnp
from jax import lax
from jax.experimental import pallas as pl
from jax.experimental.pallas import tpu as pltpu
from jax.experimental.pallas import tpu_sc as plsc

D_MODEL = 64
SCALE = math.sqrt(D_MODEL)

NC = 2
NS = 16
NW = NC * NS

B_TOTAL = 1024 * 200
B_PER_W = B_TOTAL // NW
CHUNK = 128
NCHUNK = B_PER_W // CHUNK
NBUF = 5
NGROUP = NCHUNK // NBUF


def _emb_kernel(table_hbm, x_hbm, out_hbm, idx_v, *scr):
    gbufs = scr[0:NBUF]
    obufs = scr[NBUF:2 * NBUF]
    gsems = scr[2 * NBUF:3 * NBUF]
    psems = scr[3 * NBUF:4 * NBUF]

    wid = lax.axis_index("s") * NC + lax.axis_index("c")
    base = wid * B_PER_W

    pltpu.sync_copy(x_hbm.at[wid], idx_v)

    def start_gather(j, b):
        pltpu.async_copy(table_hbm.at[idx_v.at[j]], gbufs[b], gsems[b])

    def wait_gather(b):
        pltpu.make_async_copy(table_hbm.at[idx_v.at[0]], gbufs[b], gsems[b]).wait()

    def start_put(j, b):
        pltpu.async_copy(obufs[b], out_hbm.at[pl.ds(base + j * CHUNK, CHUNK)], psems[b])

    def wait_put(b):
        pltpu.make_async_copy(obufs[b], out_hbm.at[pl.ds(base, CHUNK)], psems[b]).wait()

    def mul_chunk(b):
        gb, ob = gbufs[b], obufs[b]

        def body(r):
            for d in range(4):
                sl = pl.ds(d * 16, 16)
                ob[r, sl] = gb[r, sl] * SCALE

        pl.loop(0, CHUNK, unroll=4)(body)

    for b in range(NBUF):
        start_gather(b, b)

    for b in range(NBUF):
        wait_gather(b)
        mul_chunk(b)
        start_put(b, b)
        start_gather(NBUF + b, b)

    def round_body(g):
        for b in range(NBUF):
            j = g * NBUF + b
            wait_gather(b)
            wait_put(b)
            mul_chunk(b)
            start_put(j, b)
            start_gather(j + NBUF, b)

    pl.loop(1, NGROUP - 1)(round_body)

    for b in range(NBUF):
        j = (NGROUP - 1) * NBUF + b
        wait_gather(b)
        wait_put(b)
        mul_chunk(b)
        start_put(j, b)

    for b in range(NBUF):
        wait_put(b)


@jax.jit
def kernel(x, table):
    mesh = plsc.VectorSubcoreMesh(core_axis_name="c", subcore_axis_name="s")
    x_flat = x.reshape(NW, NCHUNK, CHUNK).astype(jnp.int32)
    scratch = (
        [pltpu.VMEM((NCHUNK, CHUNK), jnp.int32)]
        + [pltpu.VMEM((CHUNK, D_MODEL), jnp.float32) for _ in range(2 * NBUF)]
        + [pltpu.SemaphoreType.DMA for _ in range(2 * NBUF)]
    )
    run = pl.kernel(
        _emb_kernel,
        out_type=jax.ShapeDtypeStruct((B_TOTAL, D_MODEL), jnp.float32),
        mesh=mesh,
        scratch_types=scratch,
        compiler_params=pltpu.CompilerParams(use_tc_tiling_on_sc=False),
    )
    out = run(table, x_flat)
    return out.reshape(x.shape[0], x.shape[1], D_MODEL)
